# Initial kernel scaffold; baseline (speedup 1.0000x reference)
import jax
import jax.numpy as jnp
from jax.experimental import pallas as pl


def kernel(flat, segment_ids, calib_kernel, lattice_kernel, mid_kernel, final_kernel):
    raise NotImplementedError("write your pallas kernel here")


# speedup vs baseline: 5.1367x; 5.1367x over previous
import functools

import jax
import jax.numpy as jnp
from jax.experimental import pallas as pl
from jax.experimental.pallas import tpu as pltpu

B = 16
F = 6
M = 8
K = 10
BT = 4096


def _tc_body(xT_ref, seg_ref, cal_ref, lat_ref, midk_ref, fin_ref,
             out_ref, acc_ref, cnt_ref):
    pid = pl.program_id(0)
    nblk = pl.num_programs(0)

    x = xT_ref[...]
    seg = seg_ref[...]

    iota_b = jax.lax.broadcasted_iota(jnp.int32, (BT, B), 1)
    onehot = (seg == iota_b).astype(jnp.float32)

    x9 = x * 9.0
    ws = [jnp.clip(x9 - float(k), 0.0, 1.0) for k in range(K - 1)]

    cal = cal_ref[...]
    lat = lat_ref[...]

    touts = []
    for m in range(M):
        cm = jnp.zeros((F, BT), jnp.float32) + cal[:, m * K:m * K + 1]
        for k in range(K - 1):
            cm = cm + ws[k] * cal[:, m * K + k + 1:m * K + k + 2]
        cm = jnp.clip(cm, 0.0, 1.0)
        latcol = lat[:, m:m + 1]
        half = (2 ** F) // 2
        x0 = cm[0:1, :]
        vals = latcol[:half] + (latcol[half:] - latcol[:half]) * x0
        for d in range(1, F):
            half //= 2
            xd = cm[d:d + 1, :]
            vals = vals[:half] + (vals[half:] - vals[:half]) * xd
        touts.append(vals)

    tok = jnp.concatenate(touts, axis=0)
    psum = jnp.dot(tok, onehot, preferred_element_type=jnp.float32)
    pcnt = jnp.sum(onehot, axis=0, keepdims=True)

    @pl.when(pid == 0)
    def _():
        acc_ref[...] = psum
        cnt_ref[...] = pcnt

    @pl.when(pid > 0)
    def _():
        acc_ref[...] += psum
        cnt_ref[...] += pcnt

    @pl.when(pid == nblk - 1)
    def _():
        agg = acc_ref[...] / jnp.maximum(cnt_ref[...], 1.0)
        midk = midk_ref[...]
        mid = jnp.zeros((M, B), jnp.float32) + midk[:, 0:1]
        for k in range(K - 1):
            kp = -1.0 + 2.0 * k / (K - 1)
            wmk = jnp.clip((agg - kp) * ((K - 1) / 2.0), 0.0, 1.0)
            mid = mid + wmk * midk[:, k + 1:k + 2]
        mid = jnp.clip(mid, 0.0, 1.0)
        fin = fin_ref[...]
        half = (2 ** M) // 2
        x0 = mid[0:1, :]
        vals = fin[:half] + (fin[half:] - fin[:half]) * x0
        for d in range(1, M):
            half //= 2
            xd = mid[d:d + 1, :]
            vals = vals[:half] + (vals[half:] - vals[:half]) * xd
        out_ref[...] = vals


@functools.partial(jax.jit, static_argnums=())
def _run_tc(xT, seg2, cal2, lat2, midk, fin2):
    T = xT.shape[1]
    nblk = T // BT
    grid = (nblk,)
    out = pl.pallas_call(
        _tc_body,
        grid=grid,
        in_specs=[
            pl.BlockSpec((F, BT), lambda i: (0, i)),
            pl.BlockSpec((BT, 1), lambda i: (i, 0)),
            pl.BlockSpec((F, M * K), lambda i: (0, 0)),
            pl.BlockSpec((2 ** F, M), lambda i: (0, 0)),
            pl.BlockSpec((M, K), lambda i: (0, 0)),
            pl.BlockSpec((2 ** M, 1), lambda i: (0, 0)),
        ],
        out_specs=pl.BlockSpec((1, B), lambda i: (0, 0)),
        out_shape=jax.ShapeDtypeStruct((1, B), jnp.float32),
        scratch_shapes=[
            pltpu.VMEM((M, B), jnp.float32),
            pltpu.VMEM((1, B), jnp.float32),
        ],
    )(xT, seg2, cal2, lat2, midk, fin2)
    return out


def kernel(flat, segment_ids, calib_kernel, lattice_kernel, mid_kernel,
           final_kernel):
    T = flat.shape[0]
    xT = flat.T
    seg2 = segment_ids.astype(jnp.int32).reshape(T, 1)
    cal2 = jnp.transpose(calib_kernel, (1, 0, 2)).reshape(F, M * K)
    lat2 = lattice_kernel.T
    fin2 = final_kernel.reshape(2 ** M, 1)
    out = _run_tc(xT, seg2, cal2, lat2, mid_kernel, fin2)
    return out.reshape(B, 1)
